# Initial kernel scaffold; baseline (speedup 1.0000x reference)
import functools

import jax
import jax.numpy as jnp
from jax import lax
from jax.experimental import pallas as pl
from jax.experimental.pallas import tpu as pltpu
from jax.experimental.pallas import tpu_sc as plsc

_VOCAB = 1000000
_MAX_LEN = 200
_EMBED = 32
_BATCH = 4096

_NC, _NS, _LANES = 2, 16, 16
_NW = _NC * _NS
_N = _BATCH * _MAX_LEN
_RPW = _N // _NW
_CH = 1280
_NCH = _RPW // _CH


def _body(idx_hbm, vocab_hbm, pos_hbm, out_hbm, idx_v, pos_idx_v, buf, sem):
    wid = lax.axis_index("s") * _NC + lax.axis_index("c")
    wbase = wid * _RPW
    lane = lax.iota(jnp.int32, 16)

    @pl.loop(0, _NCH)
    def _chunk(c):
        base = pl.multiple_of(wbase + c * _CH, _CH)

        pltpu.sync_copy(idx_hbm.at[pl.ds(base, _CH)], idx_v)

        @pl.loop(0, _CH // _LANES)
        def _fill(j):
            v = (base + j * _LANES + lane) % _MAX_LEN
            pos_idx_v[pl.ds(j * _LANES, _LANES)] = v

        pltpu.async_copy(vocab_hbm.at[idx_v], buf, sem).wait()
        pltpu.async_copy(pos_hbm.at[pos_idx_v], buf, sem, add=True).wait()

        pltpu.sync_copy(buf, out_hbm.at[pl.ds(base, _CH)])


@jax.jit
def _run(idx_flat, vocab_table, pos_table):
    mesh = plsc.VectorSubcoreMesh(core_axis_name="c", subcore_axis_name="s")
    f = pl.kernel(
        _body,
        out_type=jax.ShapeDtypeStruct((_N, _EMBED), jnp.float32),
        mesh=mesh,
        scratch_types=[
            pltpu.VMEM((_CH,), jnp.int32),
            pltpu.VMEM((_CH,), jnp.int32),
            pltpu.VMEM((_CH, _EMBED), jnp.float32),
            pltpu.SemaphoreType.DMA,
        ],
    )
    return f(idx_flat, vocab_table, pos_table)


def kernel(inputs, vocab_table, pos_table):
    idx_flat = inputs.reshape(-1).astype(jnp.int32)
    out = _run(idx_flat, vocab_table, pos_table)
    return out.reshape(_BATCH, _MAX_LEN, _EMBED)


# speedup vs baseline: 1.1723x; 1.1723x over previous
import functools

import jax
import jax.numpy as jnp
from jax import lax
from jax.experimental import pallas as pl
from jax.experimental.pallas import tpu as pltpu
from jax.experimental.pallas import tpu_sc as plsc

_VOCAB = 1000000
_MAX_LEN = 200
_EMBED = 32
_BATCH = 4096

_NC, _NS, _LANES = 2, 16, 16
_NW = _NC * _NS
_N = _BATCH * _MAX_LEN
_RPW = _N // _NW
_CH = 1280
_NCH = _RPW // _CH


def _body(idx_hbm, vocab_hbm, pos_hbm, out_hbm, idx_v, pos_idx_v, buf, sem):
    wid = lax.axis_index("s") * _NC + lax.axis_index("c")
    wbase = wid * _RPW
    lane = lax.iota(jnp.int32, 16)

    @pl.loop(0, _NCH)
    def _chunk(c):
        base = pl.multiple_of(wbase + c * _CH, _CH)

        pltpu.sync_copy(idx_hbm.at[pl.ds(base, _CH)], idx_v)

        @pl.loop(0, _CH // _LANES)
        def _fill(j):
            v = (base + j * _LANES + lane) % _MAX_LEN
            pos_idx_v[pl.ds(j * _LANES, _LANES)] = v

        pltpu.async_copy(vocab_hbm.at[idx_v], buf, sem).wait()
        pltpu.async_copy(pos_hbm.at[pos_idx_v], buf, sem, add=True).wait()

        pltpu.sync_copy(buf, out_hbm.at[pl.ds(base, _CH)])


@jax.jit
def _run(idx_flat, vocab_table, pos_table):
    mesh = plsc.VectorSubcoreMesh(core_axis_name="c", subcore_axis_name="s")
    f = pl.kernel(
        _body,
        out_type=jax.ShapeDtypeStruct((_N, _EMBED), jnp.float32),
        mesh=mesh,
        scratch_types=[
            pltpu.VMEM((_CH,), jnp.int32),
            pltpu.VMEM((_CH,), jnp.int32),
            pltpu.VMEM((_CH, _EMBED), jnp.float32),
            pltpu.SemaphoreType.DMA,
        ],
        compiler_params=pltpu.CompilerParams(use_tc_tiling_on_sc=False),
    )
    return f(idx_flat, vocab_table, pos_table)


def kernel(inputs, vocab_table, pos_table):
    idx_flat = inputs.reshape(-1).astype(jnp.int32)
    out = _run(idx_flat, vocab_table, pos_table)
    return out.reshape(_BATCH, _MAX_LEN, _EMBED)


# speedup vs baseline: 1.1775x; 1.0045x over previous
import jax
import jax.numpy as jnp
from jax import lax
from jax.experimental import pallas as pl
from jax.experimental.pallas import tpu as pltpu
from jax.experimental.pallas import tpu_sc as plsc

_MAX_LEN = 200
_EMBED = 32
_BATCH = 4096

_NC, _NS, _LANES = 2, 16, 16
_NW = _NC * _NS
_N = _BATCH * _MAX_LEN
_RPW = _N // _NW
_CH = 1600
_NCH = _RPW // _CH


def _body(idx_hbm, vocab_hbm, pos_hbm, out_hbm,
          idx_v, pos_idx_v, buf, si, sg, sa, sw):
    wid = lax.axis_index("s") * _NC + lax.axis_index("c")
    wbase = wid * _RPW
    lane = lax.iota(jnp.int32, 16)

    for j in range(_CH // _LANES):
        pos_idx_v[pl.ds(j * _LANES, _LANES)] = (j * _LANES + lane) % _MAX_LEN

    def idx_copy(c):
        p = c % 2
        return pltpu.async_copy(
            idx_hbm.at[pl.ds(wbase + c * _CH, _CH)], idx_v.at[p], si.at[p])

    def gather(c):
        p = c % 2
        return pltpu.async_copy(vocab_hbm.at[idx_v.at[p]], buf.at[p], sg.at[p])

    def pos_add(c):
        p = c % 2
        return pltpu.async_copy(pos_hbm.at[pos_idx_v], buf.at[p], sa.at[p],
                                add=True)

    def writeback(c):
        p = c % 2
        return pltpu.async_copy(
            buf.at[p], out_hbm.at[pl.ds(wbase + c * _CH, _CH)], sw.at[p])

    idx_copy(0).wait()
    d_g = {0: gather(0)}
    d_i, d_a, d_w = {}, {}, {}

    for c in range(_NCH):
        if c + 1 < _NCH:
            d_i[c + 1] = idx_copy(c + 1)
        d_g[c].wait()
        d_a[c] = pos_add(c)
        if c + 1 < _NCH:
            d_i[c + 1].wait()
            if c - 1 >= 0:
                d_w[c - 1].wait()
            d_g[c + 1] = gather(c + 1)
        d_a[c].wait()
        d_w[c] = writeback(c)

    d_w[_NCH - 2].wait()
    d_w[_NCH - 1].wait()


@jax.jit
def _run(idx_flat, vocab_table, pos_table):
    mesh = plsc.VectorSubcoreMesh(core_axis_name="c", subcore_axis_name="s")
    f = pl.kernel(
        _body,
        out_type=jax.ShapeDtypeStruct((_N, _EMBED), jnp.float32),
        mesh=mesh,
        scratch_types=[
            pltpu.VMEM((2, _CH), jnp.int32),
            pltpu.VMEM((_CH,), jnp.int32),
            pltpu.VMEM((2, _CH, _EMBED), jnp.float32),
            pltpu.SemaphoreType.DMA((2,)),
            pltpu.SemaphoreType.DMA((2,)),
            pltpu.SemaphoreType.DMA((2,)),
            pltpu.SemaphoreType.DMA((2,)),
        ],
        compiler_params=pltpu.CompilerParams(use_tc_tiling_on_sc=False),
    )
    return f(idx_flat, vocab_table, pos_table)


def kernel(inputs, vocab_table, pos_table):
    idx_flat = inputs.reshape(-1).astype(jnp.int32)
    out = _run(idx_flat, vocab_table, pos_table)
    return out.reshape(_BATCH, _MAX_LEN, _EMBED)


# speedup vs baseline: 1.4938x; 1.2686x over previous
import jax
import jax.numpy as jnp
from jax import lax
from jax.experimental import pallas as pl
from jax.experimental.pallas import tpu as pltpu
from jax.experimental.pallas import tpu_sc as plsc

_MAX_LEN = 200
_EMBED = 32
_BATCH = 4096

_NC, _NS, _LANES = 2, 16, 16
_NW = _NC * _NS
_N = _BATCH * _MAX_LEN
_RPW = _N // _NW
_CH = 1600
_NCH = _RPW // _CH
_ROWS_PER_POS = _CH // _MAX_LEN


def _body(idx_hbm, vocab_hbm, pos_hbm, out_hbm,
          idx_v, pos_v, buf, si, sg, sw):
    wid = lax.axis_index("s") * _NC + lax.axis_index("c")
    wbase = wid * _RPW

    pltpu.sync_copy(pos_hbm, pos_v)

    def idx_copy(c):
        p = c % 2
        return pltpu.async_copy(
            idx_hbm.at[pl.ds(wbase + c * _CH, _CH)], idx_v.at[p], si.at[p])

    def gather(c):
        p = c % 2
        return pltpu.async_copy(vocab_hbm.at[idx_v.at[p]], buf.at[p],
                                sg.at[p])

    def pos_add(c):
        p = c % 2

        @pl.loop(0, _MAX_LEN)
        def _l(l):
            p0 = pos_v[l, pl.ds(0, _LANES)]
            p1 = pos_v[l, pl.ds(_LANES, _LANES)]
            for k in range(_ROWS_PER_POS):
                r = k * _MAX_LEN + l
                buf[p, r, pl.ds(0, _LANES)] += p0
                buf[p, r, pl.ds(_LANES, _LANES)] += p1

    def writeback(c):
        p = c % 2
        return pltpu.async_copy(
            buf.at[p], out_hbm.at[pl.ds(wbase + c * _CH, _CH)], sw.at[p])

    idx_copy(0).wait()
    d_g = {0: gather(0)}
    d_i, d_w = {}, {}

    for c in range(_NCH):
        if c + 1 < _NCH:
            d_i[c + 1] = idx_copy(c + 1)
        d_g[c].wait()
        if c + 1 < _NCH:
            d_i[c + 1].wait()
            if c - 1 >= 0:
                d_w[c - 1].wait()
            d_g[c + 1] = gather(c + 1)
        pos_add(c)
        d_w[c] = writeback(c)

    d_w[_NCH - 2].wait()
    d_w[_NCH - 1].wait()


@jax.jit
def _run(idx_flat, vocab_table, pos_table):
    mesh = plsc.VectorSubcoreMesh(core_axis_name="c", subcore_axis_name="s")
    f = pl.kernel(
        _body,
        out_type=jax.ShapeDtypeStruct((_N, _EMBED), jnp.float32),
        mesh=mesh,
        scratch_types=[
            pltpu.VMEM((2, _CH), jnp.int32),
            pltpu.VMEM((_MAX_LEN, _EMBED), jnp.float32),
            pltpu.VMEM((2, _CH, _EMBED), jnp.float32),
            pltpu.SemaphoreType.DMA((2,)),
            pltpu.SemaphoreType.DMA((2,)),
            pltpu.SemaphoreType.DMA((2,)),
        ],
        compiler_params=pltpu.CompilerParams(use_tc_tiling_on_sc=False),
    )
    return f(idx_flat, vocab_table, pos_table)


def kernel(inputs, vocab_table, pos_table):
    idx_flat = inputs.reshape(-1).astype(jnp.int32)
    out = _run(idx_flat, vocab_table, pos_table)
    return out.reshape(_BATCH, _MAX_LEN, _EMBED)
